# Initial kernel scaffold; baseline (speedup 1.0000x reference)
import jax
import jax.numpy as jnp
from jax.experimental import pallas as pl


def kernel(pos, area_point, edge_index, weight, params):
    raise NotImplementedError("write your pallas kernel here")


# speedup vs baseline: 21.2532x; 21.2532x over previous
import functools

import jax
import jax.numpy as jnp
from jax import lax
from jax.experimental import pallas as pl
from jax.experimental.pallas import tpu as pltpu
from jax.experimental.pallas import tpu_sc as plsc

N_NODES = 50000
N_EDGES = 800000

NPAD = 50176
EPAD = 819200
NW = 32
E_PER_W = EPAD // NW
CH = 1024
KJ = CH // 128
N_IT = E_PER_W // CH
DV = 24


def _silu(x):
    return x * jax.nn.sigmoid(x)


def _make_gather(dt, interpret=False):
    mesh = plsc.VectorSubcoreMesh(core_axis_name="c", subcore_axis_name="s")

    @functools.partial(
        pl.kernel,
        out_type=(
            jax.ShapeDtypeStruct((EPAD, dt), jnp.float32),
            jax.ShapeDtypeStruct((EPAD, dt), jnp.float32),
        ),
        mesh=mesh,
        scratch_types=[
            pltpu.VMEM((KJ, 128), jnp.int32),
            pltpu.VMEM((KJ, 128), jnp.int32),
            pltpu.VMEM((CH, dt), jnp.float32),
            pltpu.VMEM((CH, dt), jnp.float32),
            pltpu.SemaphoreType.DMA,
            pltpu.SemaphoreType.DMA,
        ],
        compiler_params=pltpu.CompilerParams(use_tc_tiling_on_sc=False),
        interpret=interpret,
    )
    def gather_k(tab_hbm, row_hbm, col_hbm, outr_hbm, outc_hbm,
                 idxr_v, idxc_v, bufr_v, bufc_v, semr, semc):
        wid = lax.axis_index("s") * 2 + lax.axis_index("c")
        base0 = wid * E_PER_W

        def body(i, _):
            base = pl.multiple_of(base0 + i * CH, CH)
            blk = pl.multiple_of(base // 128, KJ)
            pltpu.sync_copy(row_hbm.at[pl.ds(blk, KJ)], idxr_v)
            pltpu.sync_copy(col_hbm.at[pl.ds(blk, KJ)], idxc_v)
            descs = []
            for j in range(KJ):
                descs.append(pltpu.async_copy(
                    tab_hbm.at[idxr_v.at[j]],
                    bufr_v.at[pl.ds(j * 128, 128)], semr))
                descs.append(pltpu.async_copy(
                    tab_hbm.at[idxc_v.at[j]],
                    bufc_v.at[pl.ds(j * 128, 128)], semc))
            for d in descs:
                d.wait()
            pltpu.sync_copy(bufr_v, outr_hbm.at[pl.ds(base, CH)])
            pltpu.sync_copy(bufc_v, outc_hbm.at[pl.ds(base, CH)])
            return 0

        lax.fori_loop(0, N_IT, body, 0)

    return gather_k


def _make_scatter(interpret=False):
    mesh = plsc.VectorSubcoreMesh(core_axis_name="c", subcore_axis_name="s")
    ZR = 392
    ROWS_PER_TILE = NPAD // 16

    @functools.partial(
        pl.kernel,
        out_type=jax.ShapeDtypeStruct((2, NPAD, DV), jnp.float32),
        mesh=mesh,
        scratch_types=[
            pltpu.VMEM((KJ, 128), jnp.int32),
            pltpu.VMEM((CH, DV), jnp.float32),
            pltpu.VMEM((ZR, DV), jnp.float32),
            pltpu.VMEM_SHARED((NPAD, DV), jnp.float32),
            pltpu.SemaphoreType.DMA,
        ],
        compiler_params=pltpu.CompilerParams(use_tc_tiling_on_sc=False),
        interpret=interpret,
    )
    def scatter_k(val_hbm, row_hbm, out_hbm, idx_v, val_v, z_v, acc_sh, sem):
        cid = lax.axis_index("c")
        sid = lax.axis_index("s")
        wid = sid * 2 + cid

        zeros16 = jnp.zeros((16,), jnp.float32)

        def zbody(i, _):
            z_v[i, pl.ds(0, 16)] = zeros16
            z_v[i, pl.ds(DV - 16, 16)] = zeros16
            return 0

        lax.fori_loop(0, ZR, zbody, 0)
        tile_base = pl.multiple_of(sid * ROWS_PER_TILE, ROWS_PER_TILE)

        def zcopy(i, _):
            pltpu.sync_copy(
                z_v, acc_sh.at[pl.ds(pl.multiple_of(tile_base + i * ZR, ZR),
                                     ZR)])
            return 0

        lax.fori_loop(0, ROWS_PER_TILE // ZR, zcopy, 0)
        plsc.subcore_barrier()

        base0 = wid * E_PER_W

        def body(i, _):
            base = pl.multiple_of(base0 + i * CH, CH)
            blk = pl.multiple_of(base // 128, KJ)
            pltpu.sync_copy(row_hbm.at[pl.ds(blk, KJ)], idx_v)
            pltpu.async_copy(val_hbm.at[pl.ds(base, CH)], val_v, sem).wait()
            for j in range(KJ):
                pltpu.sync_copy(val_v.at[pl.ds(j * 128, 128)],
                                acc_sh.at[idx_v.at[j]], add=True)
            return 0

        lax.fori_loop(0, N_IT, body, 0)
        plsc.subcore_barrier()
        pltpu.sync_copy(acc_sh.at[pl.ds(tile_base, ROWS_PER_TILE)],
                        out_hbm.at[cid, pl.ds(tile_base, ROWS_PER_TILE)])

    return scatter_k


def _edge_tc(gr, gc, wt, p, in_nf, vin, vout, interpret=False):
    dt = gr.shape[1]
    BE = 1024
    grid = EPAD // BE

    def body(gr_ref, gc_ref, wt_ref, We1, be1, We2, be2, Wc1, bc1, Wc2,
             v_ref):
        hr = gr_ref[:, :in_nf]
        hc = gc_ref[:, :in_nf]
        cr = gr_ref[:, in_nf:in_nf + 3 * vin]
        cc = gc_ref[:, in_nf:in_nf + 3 * vin]
        diff = cr - cc
        rads = [jnp.sum(diff[:, 3 * v:3 * v + 3] ** 2, axis=1, keepdims=True)
                for v in range(vin)]
        ef = jnp.concatenate([hr, hc] + rads + [wt_ref[:]], axis=1)
        m = _silu(ef @ We1[:] + be1[:])
        m = _silu(m @ We2[:] + be2[:])
        c2 = _silu(m @ Wc1[:] + bc1[:])
        phi = c2 @ Wc2[:]
        outs = [m]
        for o in range(vout):
            acc = phi[:, o * vin:o * vin + 1] * diff[:, 0:3]
            for i in range(1, vin):
                acc = acc + (phi[:, o * vin + i:o * vin + i + 1]
                             * diff[:, 3 * i:3 * i + 3])
            outs.append(acc)
        B = gr_ref.shape[0]
        outs.append(jnp.ones((B, 1), jnp.float32))
        pad = DV - 16 - 3 * vout - 1
        if pad:
            outs.append(jnp.zeros((B, pad), jnp.float32))
        v_ref[:] = jnp.concatenate(outs, axis=1)

    full = lambda a: pl.BlockSpec(a.shape, lambda i: (0,) * a.ndim)
    return pl.pallas_call(
        body,
        grid=(grid,),
        in_specs=[
            pl.BlockSpec((BE, dt), lambda i: (i, 0)),
            pl.BlockSpec((BE, dt), lambda i: (i, 0)),
            pl.BlockSpec((BE, 1), lambda i: (i, 0)),
            full(p['We1']), full(p['be1']), full(p['We2']), full(p['be2']),
            full(p['Wc1']), full(p['bc1']), full(p['Wc2']),
        ],
        out_specs=pl.BlockSpec((BE, DV), lambda i: (i, 0)),
        out_shape=jax.ShapeDtypeStruct((EPAD, DV), jnp.float32),
        interpret=interpret,
    )(gr, gc, wt, p['We1'], p['be1'], p['We2'], p['be2'],
      p['Wc1'], p['bc1'], p['Wc2'])


def _node_tc(t, s0, s1, p, in_nf, vin, vout, head=None, interpret=False):
    dt = t.shape[1]
    BN = 1024
    grid = NPAD // BN
    dt_out = 16 + 3 * vout
    dt_next = 24 if dt_out > 8 else 8
    out_d = 10 if head is not None else dt_next

    def body(*refs):
        if head is not None:
            (t_ref, s0_ref, s1_ref, Wmix, Wn1, bn1, Wn2, bn2,
             W1, b1, W2, b2, out_ref) = refs
        else:
            (t_ref, s0_ref, s1_ref, Wmix, Wn1, bn1, Wn2, bn2,
             out_ref) = refs
        agg = s0_ref[:] + s1_ref[:]
        m_agg = agg[:, :16]
        cnt = jnp.maximum(agg[:, 16 + 3 * vout:17 + 3 * vout], 1.0)
        h = t_ref[:, :in_nf]
        coord = t_ref[:, in_nf:in_nf + 3 * vin]
        couts = []
        for o in range(vout):
            acc = agg[:, 16 + 3 * o:19 + 3 * o] / cnt
            for i in range(vin):
                acc = acc + Wmix[o:o + 1, i:i + 1] * coord[:, 3 * i:3 * i + 3]
            couts.append(acc)
        hn = jnp.concatenate([h, m_agg], axis=1)
        hn = _silu(hn @ Wn1[:] + bn1[:])
        h_out = hn @ Wn2[:] + bn2[:]
        if head is not None:
            z = jnp.maximum(h_out @ W1[:] + b1[:], 0.0)
            y = z @ W2[:] + b2[:]
            ymax = jnp.max(y, axis=1, keepdims=True)
            lse = ymax + jnp.log(jnp.sum(jnp.exp(y - ymax), axis=1,
                                         keepdims=True))
            out_ref[:] = y - lse
        else:
            outs = [h_out] + couts
            pad = dt_next - dt_out
            if pad:
                outs.append(jnp.zeros((t_ref.shape[0], pad), jnp.float32))
            out_ref[:] = jnp.concatenate(outs, axis=1)

    full = lambda a: pl.BlockSpec(a.shape, lambda i: (0,) * a.ndim)
    args = [t, s0, s1, p['Wmix'], p['Wn1'], p['bn1'], p['Wn2'], p['bn2']]
    specs = [
        pl.BlockSpec((BN, dt), lambda i: (i, 0)),
        pl.BlockSpec((BN, DV), lambda i: (i, 0)),
        pl.BlockSpec((BN, DV), lambda i: (i, 0)),
        full(p['Wmix']), full(p['Wn1']), full(p['bn1']),
        full(p['Wn2']), full(p['bn2']),
    ]
    if head is not None:
        W1, b1, W2, b2 = head
        args += [W1, b1, W2, b2]
        specs += [full(W1), full(b1), full(W2), full(b2)]
    return pl.pallas_call(
        body,
        grid=(grid,),
        in_specs=specs,
        out_specs=pl.BlockSpec((BN, out_d), lambda i: (i, 0)),
        out_shape=jax.ShapeDtypeStruct((NPAD, out_d), jnp.float32),
        interpret=interpret,
    )(*args)


def _prep_params(p):
    q = dict(p)
    for k in ('be1', 'be2', 'bc1', 'bn1', 'bn2'):
        q[k] = p[k][None, :]
    return q


def kernel(pos, area_point, edge_index, weight, params):
    row = edge_index[0].astype(jnp.int32)
    col = edge_index[1].astype(jnp.int32)
    padi = jnp.full((EPAD - N_EDGES,), N_NODES, jnp.int32)
    row2d = jnp.concatenate([row, padi]).reshape(EPAD // 128, 128)
    col2d = jnp.concatenate([col, padi]).reshape(EPAD // 128, 128)
    wt = jnp.pad(weight.astype(jnp.float32), ((0, EPAD - N_EDGES), (0, 0)))

    t0 = jnp.pad(
        jnp.concatenate([area_point[:, None], pos], axis=1).astype(jnp.float32),
        ((0, NPAD - N_NODES), (0, 4)))

    convs = [
        (params['conv1'], 1, 1, 2),
        (params['conv2'], 16, 2, 2),
        (params['conv3'], 16, 2, 1),
    ]
    head = (params['W1'], params['b1'][None, :],
            params['W2'], params['b2'][None, :])

    t = t0
    for li, (cp, in_nf, vin, vout) in enumerate(convs):
        cp = _prep_params(cp)
        gk = _make_gather(t.shape[1])
        gr, gc = gk(t, row2d, col2d)
        v = _edge_tc(gr, gc, wt, cp, in_nf, vin, vout)
        sk = _make_scatter()
        s = sk(v, row2d)
        t = _node_tc(t, s[0], s[1], cp, in_nf, vin, vout,
                     head=head if li == 2 else None)
    return t[:N_NODES]


# speedup vs baseline: 24.3781x; 1.1470x over previous
import functools

import jax
import jax.numpy as jnp
from jax import lax
from jax.experimental import pallas as pl
from jax.experimental.pallas import tpu as pltpu
from jax.experimental.pallas import tpu_sc as plsc

N_NODES = 50000
N_EDGES = 800000

NPAD = 50176
EPAD = 819200
NW = 32
E_PER_W = EPAD // NW
CH = 1024
KJ = CH // 128
N_IT = E_PER_W // CH
DV = 24


def _silu(x):
    return x * jax.nn.sigmoid(x)


def _make_gather(dt, interpret=False):
    mesh = plsc.VectorSubcoreMesh(core_axis_name="c", subcore_axis_name="s")

    @functools.partial(
        pl.kernel,
        out_type=(
            jax.ShapeDtypeStruct((EPAD, dt), jnp.float32),
            jax.ShapeDtypeStruct((EPAD, dt), jnp.float32),
        ),
        mesh=mesh,
        scratch_types=[
            pltpu.VMEM((KJ, 128), jnp.int32),
            pltpu.VMEM((KJ, 128), jnp.int32),
            pltpu.VMEM((CH, dt), jnp.float32),
            pltpu.VMEM((CH, dt), jnp.float32),
            pltpu.SemaphoreType.DMA,
            pltpu.SemaphoreType.DMA,
        ],
        compiler_params=pltpu.CompilerParams(use_tc_tiling_on_sc=False),
        interpret=interpret,
    )
    def gather_k(tab_hbm, row_hbm, col_hbm, outr_hbm, outc_hbm,
                 idxr_v, idxc_v, bufr_v, bufc_v, semr, semc):
        wid = lax.axis_index("s") * 2 + lax.axis_index("c")
        base0 = wid * E_PER_W

        def body(i, _):
            base = pl.multiple_of(base0 + i * CH, CH)
            blk = pl.multiple_of(base // 128, KJ)
            pltpu.sync_copy(row_hbm.at[pl.ds(blk, KJ)], idxr_v)
            pltpu.sync_copy(col_hbm.at[pl.ds(blk, KJ)], idxc_v)
            descs = []
            for j in range(KJ):
                descs.append(pltpu.async_copy(
                    tab_hbm.at[idxr_v.at[j]],
                    bufr_v.at[pl.ds(j * 128, 128)], semr))
                descs.append(pltpu.async_copy(
                    tab_hbm.at[idxc_v.at[j]],
                    bufc_v.at[pl.ds(j * 128, 128)], semc))
            for d in descs:
                d.wait()
            pltpu.sync_copy(bufr_v, outr_hbm.at[pl.ds(base, CH)])
            pltpu.sync_copy(bufc_v, outc_hbm.at[pl.ds(base, CH)])
            return 0

        lax.fori_loop(0, N_IT, body, 0)

    return gather_k


def _make_scatter(interpret=False):
    mesh = plsc.VectorSubcoreMesh(core_axis_name="c", subcore_axis_name="s")
    ZR = 392
    ROWS_PER_TILE = NPAD // 16

    @functools.partial(
        pl.kernel,
        out_type=jax.ShapeDtypeStruct((2, NPAD, DV), jnp.float32),
        mesh=mesh,
        scratch_types=[
            pltpu.VMEM((KJ, 128), jnp.int32),
            pltpu.VMEM((CH, DV), jnp.float32),
            pltpu.VMEM((ZR, DV), jnp.float32),
            pltpu.VMEM_SHARED((NPAD, DV), jnp.float32),
            pltpu.SemaphoreType.DMA,
        ],
        compiler_params=pltpu.CompilerParams(use_tc_tiling_on_sc=False),
        interpret=interpret,
    )
    def scatter_k(val_hbm, row_hbm, out_hbm, idx_v, val_v, z_v, acc_sh, sem):
        cid = lax.axis_index("c")
        sid = lax.axis_index("s")
        wid = sid * 2 + cid

        zeros16 = jnp.zeros((16,), jnp.float32)

        def zbody(i, _):
            z_v[i, pl.ds(0, 16)] = zeros16
            z_v[i, pl.ds(DV - 16, 16)] = zeros16
            return 0

        lax.fori_loop(0, ZR, zbody, 0)
        tile_base = pl.multiple_of(sid * ROWS_PER_TILE, ROWS_PER_TILE)

        def zcopy(i, _):
            pltpu.sync_copy(
                z_v, acc_sh.at[pl.ds(pl.multiple_of(tile_base + i * ZR, ZR),
                                     ZR)])
            return 0

        lax.fori_loop(0, ROWS_PER_TILE // ZR, zcopy, 0)
        plsc.subcore_barrier()

        base0 = wid * E_PER_W

        def body(i, _):
            base = pl.multiple_of(base0 + i * CH, CH)
            blk = pl.multiple_of(base // 128, KJ)
            pltpu.sync_copy(row_hbm.at[pl.ds(blk, KJ)], idx_v)
            pltpu.async_copy(val_hbm.at[pl.ds(base, CH)], val_v, sem).wait()
            for j in range(KJ):
                pltpu.sync_copy(val_v.at[pl.ds(j * 128, 128)],
                                acc_sh.at[idx_v.at[j]], add=True)
            return 0

        lax.fori_loop(0, N_IT, body, 0)
        plsc.subcore_barrier()
        pltpu.sync_copy(acc_sh.at[pl.ds(tile_base, ROWS_PER_TILE)],
                        out_hbm.at[cid, pl.ds(tile_base, ROWS_PER_TILE)])

    return scatter_k


def _edge_consts(p, dt, in_nf, vin, vout):
    import numpy as np
    K = 3 * vin * vout
    We1 = p['We1']
    zpad = jnp.zeros((dt - in_nf, 16), jnp.float32)
    Ap = jnp.concatenate([We1[:in_nf], zpad], axis=0)
    Bp = jnp.concatenate([We1[in_nf:2 * in_nf], zpad], axis=0)
    Cp = jnp.zeros((dt, 16), jnp.float32)
    for v in range(vin):
        for k in range(3):
            Cp = Cp.at[in_nf + 3 * v + k].set(We1[2 * in_nf + v])
    D = We1[2 * in_nf + vin:2 * in_nf + vin + 1]
    X = np.zeros((vin * vout, K), np.float32)
    for o in range(vout):
        for i in range(vin):
            for k in range(3):
                X[o * vin + i, 3 * (o * vin + i) + k] = 1.0
    WX = p['Wc2'] @ jnp.asarray(X)
    Y = np.zeros((dt, K), np.float32)
    for o in range(vout):
        for i in range(vin):
            for k in range(3):
                Y[in_nf + 3 * i + k, 3 * (o * vin + i) + k] = 1.0
    Z = np.zeros((K, DV), np.float32)
    for o in range(vout):
        for i in range(vin):
            for k in range(3):
                Z[3 * (o * vin + i) + k, 16 + 3 * o + k] = 1.0
    Pm = np.zeros((16, DV), np.float32)
    Pm[:16, :16] = np.eye(16, dtype=np.float32)
    brow = np.zeros((1, DV), np.float32)
    brow[0, 16 + 3 * vout] = 1.0
    return (Ap, Bp, Cp, D, WX, jnp.asarray(Y), jnp.asarray(Z),
            jnp.asarray(Pm), jnp.asarray(brow))


def _edge_tc(gr, gc, wt, p, in_nf, vin, vout, interpret=False):
    dt = gr.shape[1]
    BE = 2048
    grid = EPAD // BE
    Ap, Bp, Cp, D, WX, Yp, Zp, Pm, brow = _edge_consts(p, dt, in_nf, vin,
                                                       vout)

    def body(gr_ref, gc_ref, wt_ref, Ap_r, Bp_r, Cp_r, D_r, be1, We2, be2,
             Wc1, bc1, WX_r, Yp_r, Zp_r, Pm_r, brow_r, v_ref):
        g = gr_ref[:] - gc_ref[:]
        m = _silu(gr_ref[:] @ Ap_r[:] + gc_ref[:] @ Bp_r[:]
                  + (g * g) @ Cp_r[:] + wt_ref[:] @ D_r[:] + be1[:])
        m = _silu(m @ We2[:] + be2[:])
        c2 = _silu(m @ Wc1[:] + bc1[:])
        prod = (c2 @ WX_r[:]) * (g @ Yp_r[:])
        v_ref[:] = m @ Pm_r[:] + prod @ Zp_r[:] + brow_r[:]

    full = lambda a: pl.BlockSpec(a.shape, lambda i: (0,) * a.ndim)
    return pl.pallas_call(
        body,
        grid=(grid,),
        in_specs=[
            pl.BlockSpec((BE, dt), lambda i: (i, 0)),
            pl.BlockSpec((BE, dt), lambda i: (i, 0)),
            pl.BlockSpec((BE, 1), lambda i: (i, 0)),
            full(Ap), full(Bp), full(Cp), full(D), full(p['be1']),
            full(p['We2']), full(p['be2']), full(p['Wc1']), full(p['bc1']),
            full(WX), full(Yp), full(Zp), full(Pm), full(brow),
        ],
        out_specs=pl.BlockSpec((BE, DV), lambda i: (i, 0)),
        out_shape=jax.ShapeDtypeStruct((EPAD, DV), jnp.float32),
        interpret=interpret,
    )(gr, gc, wt, Ap, Bp, Cp, D, p['be1'], p['We2'], p['be2'],
      p['Wc1'], p['bc1'], WX, Yp, Zp, Pm, brow)


def _node_tc(t, s0, s1, p, in_nf, vin, vout, head=None, interpret=False):
    dt = t.shape[1]
    BN = 1024
    grid = NPAD // BN
    dt_out = 16 + 3 * vout
    dt_next = 24 if dt_out > 8 else 8
    out_d = 10 if head is not None else dt_next

    def body(*refs):
        if head is not None:
            (t_ref, s0_ref, s1_ref, Wmix, Wn1, bn1, Wn2, bn2,
             W1, b1, W2, b2, out_ref) = refs
        else:
            (t_ref, s0_ref, s1_ref, Wmix, Wn1, bn1, Wn2, bn2,
             out_ref) = refs
        agg = s0_ref[:] + s1_ref[:]
        m_agg = agg[:, :16]
        cnt = jnp.maximum(agg[:, 16 + 3 * vout:17 + 3 * vout], 1.0)
        h = t_ref[:, :in_nf]
        coord = t_ref[:, in_nf:in_nf + 3 * vin]
        couts = []
        for o in range(vout):
            acc = agg[:, 16 + 3 * o:19 + 3 * o] / cnt
            for i in range(vin):
                acc = acc + Wmix[o:o + 1, i:i + 1] * coord[:, 3 * i:3 * i + 3]
            couts.append(acc)
        hn = jnp.concatenate([h, m_agg], axis=1)
        hn = _silu(hn @ Wn1[:] + bn1[:])
        h_out = hn @ Wn2[:] + bn2[:]
        if head is not None:
            z = jnp.maximum(h_out @ W1[:] + b1[:], 0.0)
            y = z @ W2[:] + b2[:]
            ymax = jnp.max(y, axis=1, keepdims=True)
            lse = ymax + jnp.log(jnp.sum(jnp.exp(y - ymax), axis=1,
                                         keepdims=True))
            out_ref[:] = y - lse
        else:
            outs = [h_out] + couts
            pad = dt_next - dt_out
            if pad:
                outs.append(jnp.zeros((t_ref.shape[0], pad), jnp.float32))
            out_ref[:] = jnp.concatenate(outs, axis=1)

    full = lambda a: pl.BlockSpec(a.shape, lambda i: (0,) * a.ndim)
    args = [t, s0, s1, p['Wmix'], p['Wn1'], p['bn1'], p['Wn2'], p['bn2']]
    specs = [
        pl.BlockSpec((BN, dt), lambda i: (i, 0)),
        pl.BlockSpec((BN, DV), lambda i: (i, 0)),
        pl.BlockSpec((BN, DV), lambda i: (i, 0)),
        full(p['Wmix']), full(p['Wn1']), full(p['bn1']),
        full(p['Wn2']), full(p['bn2']),
    ]
    if head is not None:
        W1, b1, W2, b2 = head
        args += [W1, b1, W2, b2]
        specs += [full(W1), full(b1), full(W2), full(b2)]
    return pl.pallas_call(
        body,
        grid=(grid,),
        in_specs=specs,
        out_specs=pl.BlockSpec((BN, out_d), lambda i: (i, 0)),
        out_shape=jax.ShapeDtypeStruct((NPAD, out_d), jnp.float32),
        interpret=interpret,
    )(*args)


def _prep_params(p):
    q = dict(p)
    for k in ('be1', 'be2', 'bc1', 'bn1', 'bn2'):
        q[k] = p[k][None, :]
    return q


def kernel(pos, area_point, edge_index, weight, params):
    row = edge_index[0].astype(jnp.int32)
    col = edge_index[1].astype(jnp.int32)
    padi = jnp.full((EPAD - N_EDGES,), N_NODES, jnp.int32)
    row2d = jnp.concatenate([row, padi]).reshape(EPAD // 128, 128)
    col2d = jnp.concatenate([col, padi]).reshape(EPAD // 128, 128)
    wt = jnp.pad(weight.astype(jnp.float32), ((0, EPAD - N_EDGES), (0, 0)))

    t0 = jnp.pad(
        jnp.concatenate([area_point[:, None], pos], axis=1).astype(jnp.float32),
        ((0, NPAD - N_NODES), (0, 4)))

    convs = [
        (params['conv1'], 1, 1, 2),
        (params['conv2'], 16, 2, 2),
        (params['conv3'], 16, 2, 1),
    ]
    head = (params['W1'], params['b1'][None, :],
            params['W2'], params['b2'][None, :])

    t = t0
    for li, (cp, in_nf, vin, vout) in enumerate(convs):
        cp = _prep_params(cp)
        gk = _make_gather(t.shape[1])
        gr, gc = gk(t, row2d, col2d)
        v = _edge_tc(gr, gc, wt, cp, in_nf, vin, vout)
        sk = _make_scatter()
        s = sk(v, row2d)
        t = _node_tc(t, s[0], s[1], cp, in_nf, vin, vout,
                     head=head if li == 2 else None)
    return t[:N_NODES]


# speedup vs baseline: 46.8588x; 1.9222x over previous
import functools

import jax
import jax.numpy as jnp
from jax import lax
from jax.experimental import pallas as pl
from jax.experimental.pallas import tpu as pltpu
from jax.experimental.pallas import tpu_sc as plsc

N_NODES = 50000
N_EDGES = 800000

NPAD = 50176
EPAD = 819200
NW = 32
E_PER_W = EPAD // NW
CH = 1024
KJ = CH // 128
N_IT = E_PER_W // CH
DT = 32
DV = 32


def _silu(x):
    return x * jax.nn.sigmoid(x)


def _make_gather(interpret=False):
    mesh = plsc.VectorSubcoreMesh(core_axis_name="c", subcore_axis_name="s")

    @functools.partial(
        pl.kernel,
        out_type=(
            jax.ShapeDtypeStruct((EPAD, DT), jnp.float32),
            jax.ShapeDtypeStruct((EPAD, DT), jnp.float32),
        ),
        mesh=mesh,
        scratch_types=[
            pltpu.VMEM((KJ, 128), jnp.int32),
            pltpu.VMEM((KJ, 128), jnp.int32),
            pltpu.VMEM((CH, DT), jnp.float32),
            pltpu.VMEM((CH, DT), jnp.float32),
            pltpu.SemaphoreType.DMA,
            pltpu.SemaphoreType.DMA,
        ],
        compiler_params=pltpu.CompilerParams(use_tc_tiling_on_sc=False),
        interpret=interpret,
    )
    def gather_k(tab_hbm, row_hbm, col_hbm, outr_hbm, outc_hbm,
                 idxr_v, idxc_v, bufr_v, bufc_v, semr, semc):
        wid = lax.axis_index("s") * 2 + lax.axis_index("c")
        base0 = wid * E_PER_W

        def body(i, _):
            base = pl.multiple_of(base0 + i * CH, CH)
            blk = pl.multiple_of(base // 128, KJ)
            pltpu.sync_copy(row_hbm.at[pl.ds(blk, KJ)], idxr_v)
            pltpu.sync_copy(col_hbm.at[pl.ds(blk, KJ)], idxc_v)
            descs = []
            for j in range(KJ):
                descs.append(pltpu.async_copy(
                    tab_hbm.at[idxr_v.at[j]],
                    bufr_v.at[pl.ds(j * 128, 128)], semr))
                descs.append(pltpu.async_copy(
                    tab_hbm.at[idxc_v.at[j]],
                    bufc_v.at[pl.ds(j * 128, 128)], semc))
            for d in descs:
                d.wait()
            pltpu.sync_copy(bufr_v, outr_hbm.at[pl.ds(base, CH)])
            pltpu.sync_copy(bufc_v, outc_hbm.at[pl.ds(base, CH)])
            return 0

        lax.fori_loop(0, N_IT, body, 0)

    return gather_k


def _make_scatter(interpret=False):
    mesh = plsc.VectorSubcoreMesh(core_axis_name="c", subcore_axis_name="s")
    CHS = 512
    KJS = CHS // 128
    N_ITS = E_PER_W // CHS
    ZR = 196
    ROWS_PER_TILE = NPAD // 16

    @functools.partial(
        pl.kernel,
        out_type=jax.ShapeDtypeStruct((2, NPAD, DV), jnp.float32),
        mesh=mesh,
        scratch_types=[
            pltpu.VMEM((KJS, 128), jnp.int32),
            pltpu.VMEM((CHS, DV), jnp.float32),
            pltpu.VMEM((ZR, DV), jnp.float32),
            pltpu.VMEM_SHARED((NPAD, DV), jnp.float32),
            pltpu.SemaphoreType.DMA,
        ],
        compiler_params=pltpu.CompilerParams(use_tc_tiling_on_sc=False),
        interpret=interpret,
    )
    def scatter_k(val_hbm, row_hbm, out_hbm, idx_v, val_v, z_v, acc_sh, sem):
        cid = lax.axis_index("c")
        sid = lax.axis_index("s")
        wid = sid * 2 + cid

        zeros16 = jnp.zeros((16,), jnp.float32)

        def zbody(i, _):
            z_v[i, pl.ds(0, 16)] = zeros16
            z_v[i, pl.ds(DV - 16, 16)] = zeros16
            return 0

        lax.fori_loop(0, ZR, zbody, 0)
        tile_base = pl.multiple_of(sid * ROWS_PER_TILE, ROWS_PER_TILE)

        def zcopy(i, _):
            pltpu.sync_copy(
                z_v, acc_sh.at[pl.ds(pl.multiple_of(tile_base + i * ZR, ZR),
                                     ZR)])
            return 0

        lax.fori_loop(0, ROWS_PER_TILE // ZR, zcopy, 0)
        plsc.subcore_barrier()

        base0 = wid * E_PER_W

        def body(i, _):
            base = pl.multiple_of(base0 + i * CHS, CHS)
            blk = pl.multiple_of(base // 128, KJS)
            pltpu.sync_copy(row_hbm.at[pl.ds(blk, KJS)], idx_v)
            pltpu.async_copy(val_hbm.at[pl.ds(base, CHS)], val_v, sem).wait()
            for j in range(KJS):
                pltpu.sync_copy(val_v.at[pl.ds(j * 128, 128)],
                                acc_sh.at[idx_v.at[j]], add=True)
            return 0

        lax.fori_loop(0, N_ITS, body, 0)
        plsc.subcore_barrier()
        pltpu.sync_copy(acc_sh.at[pl.ds(tile_base, ROWS_PER_TILE)],
                        out_hbm.at[cid, pl.ds(tile_base, ROWS_PER_TILE)])

    return scatter_k


def _edge_consts(p, in_nf, vin, vout):
    import numpy as np
    K = 3 * vin * vout
    We1 = p['We1']
    Ap = jnp.zeros((DT, 16), jnp.float32).at[:in_nf].set(We1[:in_nf])
    Bp = jnp.zeros((DT, 16), jnp.float32).at[:in_nf].set(
        We1[in_nf:2 * in_nf])
    Cp = jnp.zeros((DT, 16), jnp.float32)
    for v in range(vin):
        for k in range(3):
            Cp = Cp.at[in_nf + 3 * v + k].set(We1[2 * in_nf + v])
    Drow = We1[2 * in_nf + vin:2 * in_nf + vin + 1]
    X = np.zeros((vin * vout, K), np.float32)
    Y = np.zeros((DT, K), np.float32)
    Z = np.zeros((K, DV), np.float32)
    for o in range(vout):
        for i in range(vin):
            for k in range(3):
                X[o * vin + i, 3 * (o * vin + i) + k] = 1.0
                Y[in_nf + 3 * i + k, 3 * (o * vin + i) + k] = 1.0
                Z[3 * (o * vin + i) + k, 16 + 3 * o + k] = 1.0
    WX = p['Wc2'] @ jnp.asarray(X)
    Pm = np.zeros((16, DV), np.float32)
    Pm[:16, :16] = np.eye(16, dtype=np.float32)
    brow = np.zeros((1, DV), np.float32)
    brow[0, 16 + 3 * vout] = 1.0

    I4 = jnp.eye(4, dtype=jnp.float32)
    kr = lambda a: jnp.kron(I4, jnp.asarray(a))
    t4 = lambda a: jnp.tile(jnp.asarray(a), (1, 4))
    return dict(
        Ap=kr(Ap), Bp=kr(Bp), Cp=kr(Cp),
        D=t4(Drow), be1=t4(p['be1'][None, :]),
        We2=kr(p['We2']), Wc1=kr(p['Wc1']),
        be2=t4(p['be2'][None, :]), bc1=t4(p['bc1'][None, :]),
        WX=kr(WX), Yp=kr(jnp.asarray(Y)),
        Pm=kr(Pm), Zp=kr(jnp.asarray(Z)), brow=t4(brow),
    )


def _edge_tc(grp, gcp, wtr, c, interpret=False):
    BE = 4096
    BP = BE // 4
    grid = EPAD // BE

    def body(gr_ref, gc_ref, wt_ref, Ap, Bp, Cp, D, be1, We2, be2, Wc1, bc1,
             WX, Yp, Pm, Zp, brow, v_ref):
        g = gr_ref[:] - gc_ref[:]
        m = _silu(gr_ref[:] @ Ap[:] + gc_ref[:] @ Bp[:] + (g * g) @ Cp[:]
                  + wt_ref[:] * D[:] + be1[:])
        m = _silu(m @ We2[:] + be2[:])
        c2 = _silu(m @ Wc1[:] + bc1[:])
        prod = (c2 @ WX[:]) * (g @ Yp[:])
        v_ref[:] = m @ Pm[:] + prod @ Zp[:] + brow[:]

    full = lambda a: pl.BlockSpec(a.shape, lambda i: (0,) * a.ndim)
    names = ['Ap', 'Bp', 'Cp', 'D', 'be1', 'We2', 'be2', 'Wc1', 'bc1',
             'WX', 'Yp', 'Pm', 'Zp', 'brow']
    return pl.pallas_call(
        body,
        grid=(grid,),
        in_specs=[
            pl.BlockSpec((BP, 128), lambda i: (i, 0)),
            pl.BlockSpec((BP, 128), lambda i: (i, 0)),
            pl.BlockSpec((BP, 64), lambda i: (i, 0)),
        ] + [full(c[n]) for n in names],
        out_specs=pl.BlockSpec((BP, 128), lambda i: (i, 0)),
        out_shape=jax.ShapeDtypeStruct((EPAD // 4, 128), jnp.float32),
        interpret=interpret,
    )(grp, gcp, wtr, *[c[n] for n in names])


def _node_tc(t, s0, s1, p, in_nf, vin, vout, head=None, interpret=False):
    BN = 1024
    grid = NPAD // BN
    out_d = 10 if head is not None else DT

    def body(*refs):
        if head is not None:
            (t_ref, s0_ref, s1_ref, Wmix, Wn1, bn1, Wn2, bn2,
             W1, b1, W2, b2, out_ref) = refs
        else:
            (t_ref, s0_ref, s1_ref, Wmix, Wn1, bn1, Wn2, bn2,
             out_ref) = refs
        agg = s0_ref[:] + s1_ref[:]
        m_agg = agg[:, :16]
        cnt = jnp.maximum(agg[:, 16 + 3 * vout:17 + 3 * vout], 1.0)
        h = t_ref[:, :in_nf]
        coord = t_ref[:, in_nf:in_nf + 3 * vin]
        couts = []
        for o in range(vout):
            acc = agg[:, 16 + 3 * o:19 + 3 * o] / cnt
            for i in range(vin):
                acc = acc + Wmix[o:o + 1, i:i + 1] * coord[:, 3 * i:3 * i + 3]
            couts.append(acc)
        hn = jnp.concatenate([h, m_agg], axis=1)
        hn = _silu(hn @ Wn1[:] + bn1[:])
        h_out = hn @ Wn2[:] + bn2[:]
        if head is not None:
            z = jnp.maximum(h_out @ W1[:] + b1[:], 0.0)
            y = z @ W2[:] + b2[:]
            ymax = jnp.max(y, axis=1, keepdims=True)
            lse = ymax + jnp.log(jnp.sum(jnp.exp(y - ymax), axis=1,
                                         keepdims=True))
            out_ref[:] = y - lse
        else:
            outs = [h_out] + couts
            pad = DT - 16 - 3 * vout
            if pad:
                outs.append(jnp.zeros((t_ref.shape[0], pad), jnp.float32))
            out_ref[:] = jnp.concatenate(outs, axis=1)

    full = lambda a: pl.BlockSpec(a.shape, lambda i: (0,) * a.ndim)
    args = [t, s0, s1, p['Wmix'], p['Wn1'], p['bn1'][None, :], p['Wn2'],
            p['bn2'][None, :]]
    specs = [
        pl.BlockSpec((BN, DT), lambda i: (i, 0)),
        pl.BlockSpec((BN, DV), lambda i: (i, 0)),
        pl.BlockSpec((BN, DV), lambda i: (i, 0)),
    ] + [full(a) for a in args[3:]]
    if head is not None:
        W1, b1, W2, b2 = head
        args += [W1, b1, W2, b2]
        specs += [full(W1), full(b1), full(W2), full(b2)]
    return pl.pallas_call(
        body,
        grid=(grid,),
        in_specs=specs,
        out_specs=pl.BlockSpec((BN, out_d), lambda i: (i, 0)),
        out_shape=jax.ShapeDtypeStruct((NPAD, out_d), jnp.float32),
        interpret=interpret,
    )(*args)


def kernel(pos, area_point, edge_index, weight, params):
    row = edge_index[0].astype(jnp.int32)
    col = edge_index[1].astype(jnp.int32)
    padi = jnp.full((EPAD - N_EDGES,), N_NODES, jnp.int32)
    row2d = jnp.concatenate([row, padi]).reshape(EPAD // 128, 128)
    col2d = jnp.concatenate([col, padi]).reshape(EPAD // 128, 128)
    wt = jnp.pad(weight.astype(jnp.float32), ((0, EPAD - N_EDGES), (0, 0)))
    wtr = jnp.repeat(wt.reshape(EPAD // 4, 4), 16, axis=1)

    t = jnp.pad(
        jnp.concatenate([area_point[:, None], pos], axis=1).astype(jnp.float32),
        ((0, NPAD - N_NODES), (0, DT - 4)))

    convs = [
        (params['conv1'], 1, 1, 2),
        (params['conv2'], 16, 2, 2),
        (params['conv3'], 16, 2, 1),
    ]
    head = (params['W1'], params['b1'][None, :],
            params['W2'], params['b2'][None, :])

    gk = _make_gather()
    sk = _make_scatter()
    for li, (cp, in_nf, vin, vout) in enumerate(convs):
        gr, gc = gk(t, row2d, col2d)
        v = _edge_tc(gr.reshape(EPAD // 4, 128), gc.reshape(EPAD // 4, 128),
                     wtr, _edge_consts(cp, in_nf, vin, vout))
        s = sk(v.reshape(EPAD, DV), row2d)
        t = _node_tc(t, s[0], s[1], cp, in_nf, vin, vout,
                     head=head if li == 2 else None)
    return t[:N_NODES]


# speedup vs baseline: 57.2605x; 1.2220x over previous
import functools

import jax
import jax.numpy as jnp
from jax import lax
from jax.experimental import pallas as pl
from jax.experimental.pallas import tpu as pltpu
from jax.experimental.pallas import tpu_sc as plsc

N_NODES = 50000
N_EDGES = 800000

NPAD = 50176
EPAD = 819200
EH = EPAD // 2
NW = 32
E_PER_W = EH // NW
CH = 512
KJ = CH // 128
N_IT = E_PER_W // CH
DT = 32
DV = 32


def _silu(x):
    return x * jax.nn.sigmoid(x)


def _make_gather(half, interpret=False):
    mesh = plsc.VectorSubcoreMesh(core_axis_name="c", subcore_axis_name="s")

    @functools.partial(
        pl.kernel,
        out_type=(
            jax.ShapeDtypeStruct((EH, DT), jnp.float32),
            jax.ShapeDtypeStruct((EH, DT), jnp.float32),
        ),
        mesh=mesh,
        scratch_types=[
            pltpu.VMEM((KJ, 128), jnp.int32),
            pltpu.VMEM((KJ, 128), jnp.int32),
            pltpu.VMEM((CH, DT), jnp.float32),
            pltpu.VMEM((CH, DT), jnp.float32),
            pltpu.SemaphoreType.DMA,
            pltpu.SemaphoreType.DMA,
        ],
        compiler_params=pltpu.CompilerParams(use_tc_tiling_on_sc=False),
        interpret=interpret,
    )
    def gather_k(tab_hbm, row_hbm, col_hbm, outr_hbm, outc_hbm,
                 idxr_v, idxc_v, bufr_v, bufc_v, semr, semc):
        wid = lax.axis_index("s") * 2 + lax.axis_index("c")
        base0 = wid * E_PER_W

        def body(i, _):
            base = pl.multiple_of(base0 + i * CH, CH)
            blk = pl.multiple_of((half * EH + base) // 128, KJ)
            pltpu.sync_copy(row_hbm.at[pl.ds(blk, KJ)], idxr_v)
            pltpu.sync_copy(col_hbm.at[pl.ds(blk, KJ)], idxc_v)
            descs = []
            for j in range(KJ):
                descs.append(pltpu.async_copy(
                    tab_hbm.at[idxr_v.at[j]],
                    bufr_v.at[pl.ds(j * 128, 128)], semr))
                descs.append(pltpu.async_copy(
                    tab_hbm.at[idxc_v.at[j]],
                    bufc_v.at[pl.ds(j * 128, 128)], semc))
            for d in descs:
                d.wait()
            pltpu.sync_copy(bufr_v, outr_hbm.at[pl.ds(base, CH)])
            pltpu.sync_copy(bufc_v, outc_hbm.at[pl.ds(base, CH)])
            return 0

        lax.fori_loop(0, N_IT, body, 0)

    return gather_k


def _make_scatter(half, interpret=False):
    mesh = plsc.VectorSubcoreMesh(core_axis_name="c", subcore_axis_name="s")
    CHS = 512
    KJS = CHS // 128
    N_ITS = E_PER_W // CHS
    ZR = 196
    ROWS_PER_TILE = NPAD // 16

    @functools.partial(
        pl.kernel,
        out_type=jax.ShapeDtypeStruct((2, NPAD, DV), jnp.float32),
        mesh=mesh,
        scratch_types=[
            pltpu.VMEM((KJS, 128), jnp.int32),
            pltpu.VMEM((CHS, DV), jnp.float32),
            pltpu.VMEM((ZR, DV), jnp.float32),
            pltpu.VMEM_SHARED((NPAD, DV), jnp.float32),
            pltpu.SemaphoreType.DMA,
        ],
        compiler_params=pltpu.CompilerParams(use_tc_tiling_on_sc=False),
        interpret=interpret,
    )
    def scatter_k(val_hbm, row_hbm, out_hbm, idx_v, val_v, z_v, acc_sh, sem):
        cid = lax.axis_index("c")
        sid = lax.axis_index("s")
        wid = sid * 2 + cid

        zeros16 = jnp.zeros((16,), jnp.float32)

        def zbody(i, _):
            z_v[i, pl.ds(0, 16)] = zeros16
            z_v[i, pl.ds(DV - 16, 16)] = zeros16
            return 0

        lax.fori_loop(0, ZR, zbody, 0)
        tile_base = pl.multiple_of(sid * ROWS_PER_TILE, ROWS_PER_TILE)

        def zcopy(i, _):
            pltpu.sync_copy(
                z_v, acc_sh.at[pl.ds(pl.multiple_of(tile_base + i * ZR, ZR),
                                     ZR)])
            return 0

        lax.fori_loop(0, ROWS_PER_TILE // ZR, zcopy, 0)
        plsc.subcore_barrier()

        base0 = wid * E_PER_W

        def body(i, _):
            base = pl.multiple_of(base0 + i * CHS, CHS)
            blk = pl.multiple_of((half * EH + base) // 128, KJS)
            pltpu.sync_copy(row_hbm.at[pl.ds(blk, KJS)], idx_v)
            pltpu.async_copy(val_hbm.at[pl.ds(base, CHS)], val_v, sem).wait()
            for j in range(KJS):
                pltpu.sync_copy(val_v.at[pl.ds(j * 128, 128)],
                                acc_sh.at[idx_v.at[j]], add=True)
            return 0

        lax.fori_loop(0, N_ITS, body, 0)
        plsc.subcore_barrier()
        pltpu.sync_copy(acc_sh.at[pl.ds(tile_base, ROWS_PER_TILE)],
                        out_hbm.at[cid, pl.ds(tile_base, ROWS_PER_TILE)])

    return scatter_k


def _edge_consts(p, in_nf, vin, vout):
    import numpy as np
    K = 3 * vin * vout
    We1 = p['We1']
    Ap = jnp.zeros((DT, 16), jnp.float32).at[:in_nf].set(We1[:in_nf])
    Bp = jnp.zeros((DT, 16), jnp.float32).at[:in_nf].set(
        We1[in_nf:2 * in_nf])
    Cp = jnp.zeros((DT, 16), jnp.float32)
    for v in range(vin):
        for k in range(3):
            Cp = Cp.at[in_nf + 3 * v + k].set(We1[2 * in_nf + v])
    Drow = We1[2 * in_nf + vin:2 * in_nf + vin + 1]
    X = np.zeros((vin * vout, K), np.float32)
    Y = np.zeros((DT, K), np.float32)
    Z = np.zeros((K, DV), np.float32)
    for o in range(vout):
        for i in range(vin):
            for k in range(3):
                X[o * vin + i, 3 * (o * vin + i) + k] = 1.0
                Y[in_nf + 3 * i + k, 3 * (o * vin + i) + k] = 1.0
                Z[3 * (o * vin + i) + k, 16 + 3 * o + k] = 1.0
    WX = p['Wc2'] @ jnp.asarray(X)
    Pm = np.zeros((16, DV), np.float32)
    Pm[:16, :16] = np.eye(16, dtype=np.float32)
    brow = np.zeros((1, DV), np.float32)
    brow[0, 16 + 3 * vout] = 1.0

    I4 = jnp.eye(4, dtype=jnp.float32)
    kr = lambda a: jnp.kron(I4, jnp.asarray(a))
    t4 = lambda a: jnp.tile(jnp.asarray(a), (1, 4))
    return dict(
        Ap=kr(Ap), Bp=kr(Bp), Cp=kr(Cp),
        D=t4(Drow), be1=t4(p['be1'][None, :]),
        We2=kr(p['We2']), Wc1=kr(p['Wc1']),
        be2=t4(p['be2'][None, :]), bc1=t4(p['bc1'][None, :]),
        WX=kr(WX), Yp=kr(jnp.asarray(Y)),
        Pm=kr(Pm), Zp=kr(jnp.asarray(Z)), brow=t4(brow),
    )


def _edge_tc(grp, gcp, wtr, c, interpret=False):
    BE = 4096
    BP = BE // 4
    grid = EH // BE

    def body(gr_ref, gc_ref, wt_ref, Ap, Bp, Cp, D, be1, We2, be2, Wc1, bc1,
             WX, Yp, Pm, Zp, brow, v_ref):
        g = gr_ref[:] - gc_ref[:]
        m = _silu(gr_ref[:] @ Ap[:] + gc_ref[:] @ Bp[:] + (g * g) @ Cp[:]
                  + wt_ref[:] * D[:] + be1[:])
        m = _silu(m @ We2[:] + be2[:])
        c2 = _silu(m @ Wc1[:] + bc1[:])
        prod = (c2 @ WX[:]) * (g @ Yp[:])
        v_ref[:] = m @ Pm[:] + prod @ Zp[:] + brow[:]

    full = lambda a: pl.BlockSpec(a.shape, lambda i: (0,) * a.ndim)
    names = ['Ap', 'Bp', 'Cp', 'D', 'be1', 'We2', 'be2', 'Wc1', 'bc1',
             'WX', 'Yp', 'Pm', 'Zp', 'brow']
    return pl.pallas_call(
        body,
        grid=(grid,),
        in_specs=[
            pl.BlockSpec((BP, 128), lambda i: (i, 0)),
            pl.BlockSpec((BP, 128), lambda i: (i, 0)),
            pl.BlockSpec((BP, 64), lambda i: (i, 0)),
        ] + [full(c[n]) for n in names],
        out_specs=pl.BlockSpec((BP, 128), lambda i: (i, 0)),
        out_shape=jax.ShapeDtypeStruct((EH // 4, 128), jnp.float32),
        interpret=interpret,
    )(grp, gcp, wtr, *[c[n] for n in names])


def _node_tc(t, ss, p, in_nf, vin, vout, head=None, interpret=False):
    BN = 1024
    grid = NPAD // BN
    out_d = 10 if head is not None else DT

    def body(*refs):
        if head is not None:
            (t_ref, s0_ref, s1_ref, s2_ref, s3_ref, Wmix, Wn1, bn1, Wn2, bn2,
             W1, b1, W2, b2, out_ref) = refs
        else:
            (t_ref, s0_ref, s1_ref, s2_ref, s3_ref, Wmix, Wn1, bn1, Wn2, bn2,
             out_ref) = refs
        agg = (s0_ref[:] + s1_ref[:]) + (s2_ref[:] + s3_ref[:])
        m_agg = agg[:, :16]
        cnt = jnp.maximum(agg[:, 16 + 3 * vout:17 + 3 * vout], 1.0)
        h = t_ref[:, :in_nf]
        coord = t_ref[:, in_nf:in_nf + 3 * vin]
        couts = []
        for o in range(vout):
            acc = agg[:, 16 + 3 * o:19 + 3 * o] / cnt
            for i in range(vin):
                acc = acc + Wmix[o:o + 1, i:i + 1] * coord[:, 3 * i:3 * i + 3]
            couts.append(acc)
        hn = jnp.concatenate([h, m_agg], axis=1)
        hn = _silu(hn @ Wn1[:] + bn1[:])
        h_out = hn @ Wn2[:] + bn2[:]
        if head is not None:
            z = jnp.maximum(h_out @ W1[:] + b1[:], 0.0)
            y = z @ W2[:] + b2[:]
            ymax = jnp.max(y, axis=1, keepdims=True)
            lse = ymax + jnp.log(jnp.sum(jnp.exp(y - ymax), axis=1,
                                         keepdims=True))
            out_ref[:] = y - lse
        else:
            outs = [h_out] + couts
            pad = DT - 16 - 3 * vout
            if pad:
                outs.append(jnp.zeros((t_ref.shape[0], pad), jnp.float32))
            out_ref[:] = jnp.concatenate(outs, axis=1)

    full = lambda a: pl.BlockSpec(a.shape, lambda i: (0,) * a.ndim)
    args = [t] + list(ss) + [p['Wmix'], p['Wn1'], p['bn1'][None, :],
                             p['Wn2'], p['bn2'][None, :]]
    specs = [pl.BlockSpec((BN, DT), lambda i: (i, 0))] + [
        pl.BlockSpec((BN, DV), lambda i: (i, 0)) for _ in ss
    ] + [full(a) for a in args[5:]]
    if head is not None:
        W1, b1, W2, b2 = head
        args += [W1, b1, W2, b2]
        specs += [full(W1), full(b1), full(W2), full(b2)]
    return pl.pallas_call(
        body,
        grid=(grid,),
        in_specs=specs,
        out_specs=pl.BlockSpec((BN, out_d), lambda i: (i, 0)),
        out_shape=jax.ShapeDtypeStruct((NPAD, out_d), jnp.float32),
        interpret=interpret,
    )(*args)


def kernel(pos, area_point, edge_index, weight, params):
    row = edge_index[0].astype(jnp.int32)
    col = edge_index[1].astype(jnp.int32)
    padi = jnp.full((EPAD - N_EDGES,), N_NODES, jnp.int32)
    row2d = jnp.concatenate([row, padi]).reshape(EPAD // 128, 128)
    col2d = jnp.concatenate([col, padi]).reshape(EPAD // 128, 128)
    wt = jnp.pad(weight.astype(jnp.float32), ((0, EPAD - N_EDGES), (0, 0)))
    wtrs = [jnp.broadcast_to(wt[h * EH:(h + 1) * EH].reshape(EH // 4, 4, 1),
                             (EH // 4, 4, 16)).reshape(EH // 4, 64)
            for h in range(2)]

    t = jnp.pad(
        jnp.concatenate([area_point[:, None], pos], axis=1).astype(jnp.float32),
        ((0, NPAD - N_NODES), (0, DT - 4)))

    convs = [
        (params['conv1'], 1, 1, 2),
        (params['conv2'], 16, 2, 2),
        (params['conv3'], 16, 2, 1),
    ]
    head = (params['W1'], params['b1'][None, :],
            params['W2'], params['b2'][None, :])

    gks = [_make_gather(0), _make_gather(1)]
    sks = [_make_scatter(0), _make_scatter(1)]
    for li, (cp, in_nf, vin, vout) in enumerate(convs):
        consts = _edge_consts(cp, in_nf, vin, vout)
        vs = []
        for h in range(2):
            gr, gc = gks[h](t, row2d, col2d)
            vs.append(_edge_tc(gr.reshape(EH // 4, 128),
                               gc.reshape(EH // 4, 128), wtrs[h], consts))
        ss = []
        for h in range(2):
            s = sks[h](vs[h].reshape(EH, DV), row2d)
            ss += [s[0], s[1]]
        t = _node_tc(t, ss, cp, in_nf, vin, vout,
                     head=head if li == 2 else None)
    return t[:N_NODES]


# speedup vs baseline: 84.5142x; 1.4760x over previous
import functools

import jax
import jax.numpy as jnp
from jax import lax
from jax.experimental import pallas as pl
from jax.experimental.pallas import tpu as pltpu
from jax.experimental.pallas import tpu_sc as plsc

N_NODES = 50000
N_EDGES = 800000

NPAD = 50176
EPAD = 819200
EH = EPAD // 2
NW = 32
E_PER_W = EH // NW
CH = 256
KJ = CH // 128
N_IT = E_PER_W // CH
DT = 32
DV = 32


def _silu(x):
    return x * jax.nn.sigmoid(x)


def _make_gather(half, interpret=False):
    mesh = plsc.VectorSubcoreMesh(core_axis_name="c", subcore_axis_name="s")

    @functools.partial(
        pl.kernel,
        out_type=(
            jax.ShapeDtypeStruct((EH, DT), jnp.float32),
            jax.ShapeDtypeStruct((EH, DT), jnp.float32),
        ),
        mesh=mesh,
        scratch_types=[
            pltpu.VMEM((KJ, 128), jnp.int32),
            pltpu.VMEM((KJ, 128), jnp.int32),
            pltpu.VMEM((CH, DT), jnp.float32),
            pltpu.VMEM((CH, DT), jnp.float32),
            pltpu.VMEM_SHARED((NPAD, DT), jnp.float32),
            pltpu.SemaphoreType.DMA,
            pltpu.SemaphoreType.DMA,
        ],
        compiler_params=pltpu.CompilerParams(use_tc_tiling_on_sc=False),
        interpret=interpret,
    )
    def gather_k(tab_hbm, row_hbm, col_hbm, outr_hbm, outc_hbm,
                 idxr_v, idxc_v, bufr_v, bufc_v, tab_sh, semr, semc):
        wid = lax.axis_index("s") * 2 + lax.axis_index("c")
        sid = lax.axis_index("s")
        base0 = wid * E_PER_W

        TROWS = NPAD // 16
        trow = pl.multiple_of(sid * TROWS, TROWS)
        pltpu.sync_copy(tab_hbm.at[pl.ds(trow, TROWS)],
                        tab_sh.at[pl.ds(trow, TROWS)])
        plsc.subcore_barrier()

        def body(i, _):
            base = pl.multiple_of(base0 + i * CH, CH)
            blk = pl.multiple_of((half * EH + base) // 128, KJ)
            pltpu.sync_copy(row_hbm.at[pl.ds(blk, KJ)], idxr_v)
            pltpu.sync_copy(col_hbm.at[pl.ds(blk, KJ)], idxc_v)
            descs = []
            for j in range(KJ):
                descs.append(pltpu.async_copy(
                    tab_sh.at[idxr_v.at[j]],
                    bufr_v.at[pl.ds(j * 128, 128)], semr))
                descs.append(pltpu.async_copy(
                    tab_sh.at[idxc_v.at[j]],
                    bufc_v.at[pl.ds(j * 128, 128)], semc))
            for d in descs:
                d.wait()
            pltpu.sync_copy(bufr_v, outr_hbm.at[pl.ds(base, CH)])
            pltpu.sync_copy(bufc_v, outc_hbm.at[pl.ds(base, CH)])
            return 0

        lax.fori_loop(0, N_IT, body, 0)

    return gather_k


def _make_scatter(half, interpret=False):
    mesh = plsc.VectorSubcoreMesh(core_axis_name="c", subcore_axis_name="s")
    CHS = 512
    KJS = CHS // 128
    N_ITS = E_PER_W // CHS
    ZR = 196
    ROWS_PER_TILE = NPAD // 16

    @functools.partial(
        pl.kernel,
        out_type=jax.ShapeDtypeStruct((2, NPAD, DV), jnp.float32),
        mesh=mesh,
        scratch_types=[
            pltpu.VMEM((KJS, 128), jnp.int32),
            pltpu.VMEM((CHS, DV), jnp.float32),
            pltpu.VMEM((ZR, DV), jnp.float32),
            pltpu.VMEM_SHARED((NPAD, DV), jnp.float32),
            pltpu.SemaphoreType.DMA,
        ],
        compiler_params=pltpu.CompilerParams(use_tc_tiling_on_sc=False),
        interpret=interpret,
    )
    def scatter_k(val_hbm, row_hbm, out_hbm, idx_v, val_v, z_v, acc_sh, sem):
        cid = lax.axis_index("c")
        sid = lax.axis_index("s")
        wid = sid * 2 + cid

        zeros16 = jnp.zeros((16,), jnp.float32)

        def zbody(i, _):
            z_v[i, pl.ds(0, 16)] = zeros16
            z_v[i, pl.ds(DV - 16, 16)] = zeros16
            return 0

        lax.fori_loop(0, ZR, zbody, 0)
        tile_base = pl.multiple_of(sid * ROWS_PER_TILE, ROWS_PER_TILE)

        def zcopy(i, _):
            pltpu.sync_copy(
                z_v, acc_sh.at[pl.ds(pl.multiple_of(tile_base + i * ZR, ZR),
                                     ZR)])
            return 0

        lax.fori_loop(0, ROWS_PER_TILE // ZR, zcopy, 0)
        plsc.subcore_barrier()

        base0 = wid * E_PER_W

        def body(i, _):
            base = pl.multiple_of(base0 + i * CHS, CHS)
            blk = pl.multiple_of((half * EH + base) // 128, KJS)
            pltpu.sync_copy(row_hbm.at[pl.ds(blk, KJS)], idx_v)
            pltpu.async_copy(val_hbm.at[pl.ds(base, CHS)], val_v, sem).wait()
            for j in range(KJS):
                pltpu.sync_copy(val_v.at[pl.ds(j * 128, 128)],
                                acc_sh.at[idx_v.at[j]], add=True)
            return 0

        lax.fori_loop(0, N_ITS, body, 0)
        plsc.subcore_barrier()
        pltpu.sync_copy(acc_sh.at[pl.ds(tile_base, ROWS_PER_TILE)],
                        out_hbm.at[cid, pl.ds(tile_base, ROWS_PER_TILE)])

    return scatter_k


def _edge_consts(p, in_nf, vin, vout):
    import numpy as np
    K = 3 * vin * vout
    We1 = p['We1']
    Ap = jnp.zeros((DT, 16), jnp.float32).at[:in_nf].set(We1[:in_nf])
    Bp = jnp.zeros((DT, 16), jnp.float32).at[:in_nf].set(
        We1[in_nf:2 * in_nf])
    Cp = jnp.zeros((DT, 16), jnp.float32)
    for v in range(vin):
        for k in range(3):
            Cp = Cp.at[in_nf + 3 * v + k].set(We1[2 * in_nf + v])
    Drow = We1[2 * in_nf + vin:2 * in_nf + vin + 1]
    X = np.zeros((vin * vout, K), np.float32)
    Y = np.zeros((DT, K), np.float32)
    Z = np.zeros((K, DV), np.float32)
    for o in range(vout):
        for i in range(vin):
            for k in range(3):
                X[o * vin + i, 3 * (o * vin + i) + k] = 1.0
                Y[in_nf + 3 * i + k, 3 * (o * vin + i) + k] = 1.0
                Z[3 * (o * vin + i) + k, 16 + 3 * o + k] = 1.0
    WX = p['Wc2'] @ jnp.asarray(X)
    Pm = np.zeros((16, DV), np.float32)
    Pm[:16, :16] = np.eye(16, dtype=np.float32)
    brow = np.zeros((1, DV), np.float32)
    brow[0, 16 + 3 * vout] = 1.0

    I4 = jnp.eye(4, dtype=jnp.float32)
    kr = lambda a: jnp.kron(I4, jnp.asarray(a))
    t4 = lambda a: jnp.tile(jnp.asarray(a), (1, 4))
    return dict(
        Ap=kr(Ap), Bp=kr(Bp), Cp=kr(Cp),
        D=t4(Drow), be1=t4(p['be1'][None, :]),
        We2=kr(p['We2']), Wc1=kr(p['Wc1']),
        be2=t4(p['be2'][None, :]), bc1=t4(p['bc1'][None, :]),
        WX=kr(WX), Yp=kr(jnp.asarray(Y)),
        Pm=kr(Pm), Zp=kr(jnp.asarray(Z)), brow=t4(brow),
    )


def _edge_tc(grp, gcp, wtr, c, interpret=False):
    BE = 4096
    BP = BE // 4
    grid = EH // BE

    def body(gr_ref, gc_ref, wt_ref, Ap, Bp, Cp, D, be1, We2, be2, Wc1, bc1,
             WX, Yp, Pm, Zp, brow, v_ref):
        g = gr_ref[:] - gc_ref[:]
        m = _silu(gr_ref[:] @ Ap[:] + gc_ref[:] @ Bp[:] + (g * g) @ Cp[:]
                  + wt_ref[:] * D[:] + be1[:])
        m = _silu(m @ We2[:] + be2[:])
        c2 = _silu(m @ Wc1[:] + bc1[:])
        prod = (c2 @ WX[:]) * (g @ Yp[:])
        v_ref[:] = m @ Pm[:] + prod @ Zp[:] + brow[:]

    full = lambda a: pl.BlockSpec(a.shape, lambda i: (0,) * a.ndim)
    names = ['Ap', 'Bp', 'Cp', 'D', 'be1', 'We2', 'be2', 'Wc1', 'bc1',
             'WX', 'Yp', 'Pm', 'Zp', 'brow']
    return pl.pallas_call(
        body,
        grid=(grid,),
        in_specs=[
            pl.BlockSpec((BP, 128), lambda i: (i, 0)),
            pl.BlockSpec((BP, 128), lambda i: (i, 0)),
            pl.BlockSpec((BP, 64), lambda i: (i, 0)),
        ] + [full(c[n]) for n in names],
        out_specs=pl.BlockSpec((BP, 128), lambda i: (i, 0)),
        out_shape=jax.ShapeDtypeStruct((EH // 4, 128), jnp.float32),
        interpret=interpret,
    )(grp, gcp, wtr, *[c[n] for n in names])


def _node_tc(t, ss, p, in_nf, vin, vout, head=None, interpret=False):
    BN = 1024
    grid = NPAD // BN
    out_d = 10 if head is not None else DT

    def body(*refs):
        if head is not None:
            (t_ref, s0_ref, s1_ref, s2_ref, s3_ref, Wmix, Wn1, bn1, Wn2, bn2,
             W1, b1, W2, b2, out_ref) = refs
        else:
            (t_ref, s0_ref, s1_ref, s2_ref, s3_ref, Wmix, Wn1, bn1, Wn2, bn2,
             out_ref) = refs
        agg = (s0_ref[:] + s1_ref[:]) + (s2_ref[:] + s3_ref[:])
        m_agg = agg[:, :16]
        cnt = jnp.maximum(agg[:, 16 + 3 * vout:17 + 3 * vout], 1.0)
        h = t_ref[:, :in_nf]
        coord = t_ref[:, in_nf:in_nf + 3 * vin]
        couts = []
        for o in range(vout):
            acc = agg[:, 16 + 3 * o:19 + 3 * o] / cnt
            for i in range(vin):
                acc = acc + Wmix[o:o + 1, i:i + 1] * coord[:, 3 * i:3 * i + 3]
            couts.append(acc)
        hn = jnp.concatenate([h, m_agg], axis=1)
        hn = _silu(hn @ Wn1[:] + bn1[:])
        h_out = hn @ Wn2[:] + bn2[:]
        if head is not None:
            z = jnp.maximum(h_out @ W1[:] + b1[:], 0.0)
            y = z @ W2[:] + b2[:]
            ymax = jnp.max(y, axis=1, keepdims=True)
            lse = ymax + jnp.log(jnp.sum(jnp.exp(y - ymax), axis=1,
                                         keepdims=True))
            out_ref[:] = y - lse
        else:
            outs = [h_out] + couts
            pad = DT - 16 - 3 * vout
            if pad:
                outs.append(jnp.zeros((t_ref.shape[0], pad), jnp.float32))
            out_ref[:] = jnp.concatenate(outs, axis=1)

    full = lambda a: pl.BlockSpec(a.shape, lambda i: (0,) * a.ndim)
    args = [t] + list(ss) + [p['Wmix'], p['Wn1'], p['bn1'][None, :],
                             p['Wn2'], p['bn2'][None, :]]
    specs = [pl.BlockSpec((BN, DT), lambda i: (i, 0))] + [
        pl.BlockSpec((BN, DV), lambda i: (i, 0)) for _ in ss
    ] + [full(a) for a in args[5:]]
    if head is not None:
        W1, b1, W2, b2 = head
        args += [W1, b1, W2, b2]
        specs += [full(W1), full(b1), full(W2), full(b2)]
    return pl.pallas_call(
        body,
        grid=(grid,),
        in_specs=specs,
        out_specs=pl.BlockSpec((BN, out_d), lambda i: (i, 0)),
        out_shape=jax.ShapeDtypeStruct((NPAD, out_d), jnp.float32),
        interpret=interpret,
    )(*args)


def kernel(pos, area_point, edge_index, weight, params):
    row = edge_index[0].astype(jnp.int32)
    col = edge_index[1].astype(jnp.int32)
    padi = jnp.full((EPAD - N_EDGES,), N_NODES, jnp.int32)
    row2d = jnp.concatenate([row, padi]).reshape(EPAD // 128, 128)
    col2d = jnp.concatenate([col, padi]).reshape(EPAD // 128, 128)
    wt = jnp.pad(weight.astype(jnp.float32), ((0, EPAD - N_EDGES), (0, 0)))
    wtrs = [jnp.broadcast_to(wt[h * EH:(h + 1) * EH].reshape(EH // 4, 4, 1),
                             (EH // 4, 4, 16)).reshape(EH // 4, 64)
            for h in range(2)]

    t = jnp.pad(
        jnp.concatenate([area_point[:, None], pos], axis=1).astype(jnp.float32),
        ((0, NPAD - N_NODES), (0, DT - 4)))

    convs = [
        (params['conv1'], 1, 1, 2),
        (params['conv2'], 16, 2, 2),
        (params['conv3'], 16, 2, 1),
    ]
    head = (params['W1'], params['b1'][None, :],
            params['W2'], params['b2'][None, :])

    gks = [_make_gather(0), _make_gather(1)]
    sks = [_make_scatter(0), _make_scatter(1)]
    for li, (cp, in_nf, vin, vout) in enumerate(convs):
        consts = _edge_consts(cp, in_nf, vin, vout)
        vs = []
        for h in range(2):
            gr, gc = gks[h](t, row2d, col2d)
            vs.append(_edge_tc(gr.reshape(EH // 4, 128),
                               gc.reshape(EH // 4, 128), wtrs[h], consts))
        ss = []
        for h in range(2):
            s = sks[h](vs[h].reshape(EH, DV), row2d)
            ss += [s[0], s[1]]
        t = _node_tc(t, ss, cp, in_nf, vin, vout,
                     head=head if li == 2 else None)
    return t[:N_NODES]


# speedup vs baseline: 87.0463x; 1.0300x over previous
import functools

import jax
import jax.numpy as jnp
from jax import lax
from jax.experimental import pallas as pl
from jax.experimental.pallas import tpu as pltpu
from jax.experimental.pallas import tpu_sc as plsc

N_NODES = 50000
N_EDGES = 800000

NPAD = 50176
EPAD = 819200
EH = EPAD // 2
NW = 32
E_PER_W = EH // NW
CH = 256
KJ = CH // 128
N_IT = E_PER_W // CH
DT = 32
DV = 32


def _silu(x):
    return x * jax.nn.sigmoid(x)


def _make_gather(half, interpret=False):
    mesh = plsc.VectorSubcoreMesh(core_axis_name="c", subcore_axis_name="s")

    @functools.partial(
        pl.kernel,
        out_type=(
            jax.ShapeDtypeStruct((EH, DT), jnp.float32),
            jax.ShapeDtypeStruct((EH, DT), jnp.float32),
        ),
        mesh=mesh,
        scratch_types=[
            pltpu.VMEM((KJ, 128), jnp.int32),
            pltpu.VMEM((KJ, 128), jnp.int32),
            pltpu.VMEM((CH, DT), jnp.float32),
            pltpu.VMEM((CH, DT), jnp.float32),
            pltpu.VMEM_SHARED((NPAD, DT), jnp.float32),
            pltpu.SemaphoreType.DMA,
            pltpu.SemaphoreType.DMA,
        ],
        compiler_params=pltpu.CompilerParams(use_tc_tiling_on_sc=False),
        interpret=interpret,
    )
    def gather_k(tab_hbm, row_hbm, col_hbm, outr_hbm, outc_hbm,
                 idxr_v, idxc_v, bufr_v, bufc_v, tab_sh, semr, semc):
        wid = lax.axis_index("s") * 2 + lax.axis_index("c")
        sid = lax.axis_index("s")
        base0 = wid * E_PER_W

        TROWS = NPAD // 16
        trow = pl.multiple_of(sid * TROWS, TROWS)
        pltpu.sync_copy(tab_hbm.at[pl.ds(trow, TROWS)],
                        tab_sh.at[pl.ds(trow, TROWS)])
        plsc.subcore_barrier()

        def body(i, _):
            base = pl.multiple_of(base0 + i * CH, CH)
            blk = pl.multiple_of((half * EH + base) // 128, KJ)
            pltpu.sync_copy(row_hbm.at[pl.ds(blk, KJ)], idxr_v)
            pltpu.sync_copy(col_hbm.at[pl.ds(blk, KJ)], idxc_v)
            descs = []
            for j in range(KJ):
                descs.append(pltpu.async_copy(
                    tab_sh.at[idxr_v.at[j]],
                    bufr_v.at[pl.ds(j * 128, 128)], semr))
                descs.append(pltpu.async_copy(
                    tab_sh.at[idxc_v.at[j]],
                    bufc_v.at[pl.ds(j * 128, 128)], semc))
            for d in descs:
                d.wait()
            pltpu.sync_copy(bufr_v, outr_hbm.at[pl.ds(base, CH)])
            pltpu.sync_copy(bufc_v, outc_hbm.at[pl.ds(base, CH)])
            return 0

        lax.fori_loop(0, N_IT, body, 0)

    return gather_k


def _make_scatter(half, interpret=False):
    mesh = plsc.VectorSubcoreMesh(core_axis_name="c", subcore_axis_name="s")
    CHS = 512
    KJS = CHS // 128
    N_ITS = E_PER_W // CHS
    ZR = 196
    ROWS_PER_TILE = NPAD // 16

    @functools.partial(
        pl.kernel,
        out_type=jax.ShapeDtypeStruct((2, NPAD, DV), jnp.float32),
        mesh=mesh,
        scratch_types=[
            pltpu.VMEM((KJS, 128), jnp.int32),
            pltpu.VMEM((CHS, DV), jnp.float32),
            pltpu.VMEM((ZR, DV), jnp.float32),
            pltpu.VMEM_SHARED((NPAD, DV), jnp.float32),
            pltpu.SemaphoreType.DMA,
        ],
        compiler_params=pltpu.CompilerParams(use_tc_tiling_on_sc=False),
        interpret=interpret,
    )
    def scatter_k(val_hbm, row_hbm, out_hbm, idx_v, val_v, z_v, acc_sh, sem):
        cid = lax.axis_index("c")
        sid = lax.axis_index("s")
        wid = sid * 2 + cid

        zeros16 = jnp.zeros((16,), jnp.float32)

        def zbody(i, _):
            z_v[i, pl.ds(0, 16)] = zeros16
            z_v[i, pl.ds(DV - 16, 16)] = zeros16
            return 0

        lax.fori_loop(0, ZR, zbody, 0)
        tile_base = pl.multiple_of(sid * ROWS_PER_TILE, ROWS_PER_TILE)

        def zcopy(i, _):
            pltpu.sync_copy(
                z_v, acc_sh.at[pl.ds(pl.multiple_of(tile_base + i * ZR, ZR),
                                     ZR)])
            return 0

        lax.fori_loop(0, ROWS_PER_TILE // ZR, zcopy, 0)
        plsc.subcore_barrier()

        base0 = wid * E_PER_W

        def body(i, _):
            base = pl.multiple_of(base0 + i * CHS, CHS)
            blk = pl.multiple_of((half * EH + base) // 128, KJS)
            pltpu.sync_copy(row_hbm.at[pl.ds(blk, KJS)], idx_v)
            pltpu.async_copy(val_hbm.at[pl.ds(base, CHS)], val_v, sem).wait()
            for j in range(KJS):
                pltpu.sync_copy(val_v.at[pl.ds(j * 128, 128)],
                                acc_sh.at[idx_v.at[j]], add=True)
            return 0

        lax.fori_loop(0, N_ITS, body, 0)
        plsc.subcore_barrier()
        pltpu.sync_copy(acc_sh.at[pl.ds(tile_base, ROWS_PER_TILE)],
                        out_hbm.at[cid, pl.ds(tile_base, ROWS_PER_TILE)])

    return scatter_k


def _edge_consts(p, in_nf, vin, vout):
    import numpy as np
    K = 3 * vin * vout
    We1 = p['We1']
    Ap = jnp.zeros((DT, 16), jnp.float32).at[:in_nf].set(We1[:in_nf])
    Bp = jnp.zeros((DT, 16), jnp.float32).at[:in_nf].set(
        We1[in_nf:2 * in_nf])
    Cp = jnp.zeros((DT, 16), jnp.float32)
    for v in range(vin):
        for k in range(3):
            Cp = Cp.at[in_nf + 3 * v + k].set(We1[2 * in_nf + v])
    Drow = We1[2 * in_nf + vin:2 * in_nf + vin + 1]
    X = np.zeros((vin * vout, K), np.float32)
    Y = np.zeros((DT, K), np.float32)
    Z = np.zeros((K, DV), np.float32)
    for o in range(vout):
        for i in range(vin):
            for k in range(3):
                X[o * vin + i, 3 * (o * vin + i) + k] = 1.0
                Y[in_nf + 3 * i + k, 3 * (o * vin + i) + k] = 1.0
                Z[3 * (o * vin + i) + k, 16 + 3 * o + k] = 1.0
    WX = p['Wc2'] @ jnp.asarray(X)
    Pm = np.zeros((16, DV), np.float32)
    Pm[:16, :16] = np.eye(16, dtype=np.float32)
    brow = np.zeros((1, DV), np.float32)
    brow[0, 16 + 3 * vout] = 1.0

    I4 = jnp.eye(4, dtype=jnp.float32)
    kr = lambda a: jnp.kron(I4, jnp.asarray(a))
    t4 = lambda a: jnp.tile(jnp.asarray(a), (1, 4))
    return dict(
        Ap=kr(Ap), Bp=kr(Bp), Cp=kr(Cp),
        D=t4(Drow), be1=t4(p['be1'][None, :]),
        We2=kr(p['We2']), Wc1=kr(p['Wc1']),
        be2=t4(p['be2'][None, :]), bc1=t4(p['bc1'][None, :]),
        WX=kr(WX), Yp=kr(jnp.asarray(Y)),
        Pm=kr(Pm), Zp=kr(jnp.asarray(Z)), brow=t4(brow),
    )


def _edge_tc(grp, gcp, wtr, c, interpret=False):
    BE = 4096
    BP = BE // 4
    grid = EH // BE

    def body(gr_ref, gc_ref, wt_ref, Ap, Bp, Cp, D, be1, We2, be2, Wc1, bc1,
             WX, Yp, Pm, Zp, brow, v_ref):
        g = gr_ref[:] - gc_ref[:]
        m = _silu(gr_ref[:] @ Ap[:] + gc_ref[:] @ Bp[:] + (g * g) @ Cp[:]
                  + wt_ref[:].astype(jnp.float32) * D[:] + be1[:])
        m = _silu(m @ We2[:] + be2[:])
        c2 = _silu(m @ Wc1[:] + bc1[:])
        prod = (c2 @ WX[:]) * (g @ Yp[:])
        v_ref[:] = m @ Pm[:] + prod @ Zp[:] + brow[:]

    full = lambda a: pl.BlockSpec(a.shape, lambda i: (0,) * a.ndim)
    names = ['Ap', 'Bp', 'Cp', 'D', 'be1', 'We2', 'be2', 'Wc1', 'bc1',
             'WX', 'Yp', 'Pm', 'Zp', 'brow']
    return pl.pallas_call(
        body,
        grid=(grid,),
        in_specs=[
            pl.BlockSpec((BP, 128), lambda i: (i, 0)),
            pl.BlockSpec((BP, 128), lambda i: (i, 0)),
            pl.BlockSpec((BP, 64), lambda i: (i, 0)),
        ] + [full(c[n]) for n in names],
        out_specs=pl.BlockSpec((BP, 128), lambda i: (i, 0)),
        out_shape=jax.ShapeDtypeStruct((EH // 4, 128), jnp.float32),
        interpret=interpret,
    )(grp, gcp, wtr, *[c[n] for n in names])


def _node_tc(t, ss, p, in_nf, vin, vout, head=None, interpret=False):
    BN = 1024
    grid = NPAD // BN
    out_d = 10 if head is not None else DT

    def body(*refs):
        if head is not None:
            (t_ref, s0_ref, s1_ref, s2_ref, s3_ref, Wmix, Wn1, bn1, Wn2, bn2,
             W1, b1, W2, b2, out_ref) = refs
        else:
            (t_ref, s0_ref, s1_ref, s2_ref, s3_ref, Wmix, Wn1, bn1, Wn2, bn2,
             out_ref) = refs
        agg = (s0_ref[:] + s1_ref[:]) + (s2_ref[:] + s3_ref[:])
        m_agg = agg[:, :16]
        cnt = jnp.maximum(agg[:, 16 + 3 * vout:17 + 3 * vout], 1.0)
        h = t_ref[:, :in_nf]
        coord = t_ref[:, in_nf:in_nf + 3 * vin]
        couts = []
        for o in range(vout):
            acc = agg[:, 16 + 3 * o:19 + 3 * o] / cnt
            for i in range(vin):
                acc = acc + Wmix[o:o + 1, i:i + 1] * coord[:, 3 * i:3 * i + 3]
            couts.append(acc)
        hn = jnp.concatenate([h, m_agg], axis=1)
        hn = _silu(hn @ Wn1[:] + bn1[:])
        h_out = hn @ Wn2[:] + bn2[:]
        if head is not None:
            z = jnp.maximum(h_out @ W1[:] + b1[:], 0.0)
            y = z @ W2[:] + b2[:]
            ymax = jnp.max(y, axis=1, keepdims=True)
            lse = ymax + jnp.log(jnp.sum(jnp.exp(y - ymax), axis=1,
                                         keepdims=True))
            out_ref[:] = y - lse
        else:
            outs = [h_out] + couts
            pad = DT - 16 - 3 * vout
            if pad:
                outs.append(jnp.zeros((t_ref.shape[0], pad), jnp.float32))
            out_ref[:] = jnp.concatenate(outs, axis=1)

    full = lambda a: pl.BlockSpec(a.shape, lambda i: (0,) * a.ndim)
    args = [t] + list(ss) + [p['Wmix'], p['Wn1'], p['bn1'][None, :],
                             p['Wn2'], p['bn2'][None, :]]
    specs = [pl.BlockSpec((BN, DT), lambda i: (i, 0))] + [
        pl.BlockSpec((BN, DV), lambda i: (i, 0)) for _ in ss
    ] + [full(a) for a in args[5:]]
    if head is not None:
        W1, b1, W2, b2 = head
        args += [W1, b1, W2, b2]
        specs += [full(W1), full(b1), full(W2), full(b2)]
    return pl.pallas_call(
        body,
        grid=(grid,),
        in_specs=specs,
        out_specs=pl.BlockSpec((BN, out_d), lambda i: (i, 0)),
        out_shape=jax.ShapeDtypeStruct((NPAD, out_d), jnp.float32),
        interpret=interpret,
    )(*args)


def kernel(pos, area_point, edge_index, weight, params):
    row = edge_index[0].astype(jnp.int32)
    col = edge_index[1].astype(jnp.int32)
    padi = jnp.full((EPAD - N_EDGES,), N_NODES, jnp.int32)
    row2d = jnp.concatenate([row, padi]).reshape(EPAD // 128, 128)
    col2d = jnp.concatenate([col, padi]).reshape(EPAD // 128, 128)
    wt = jnp.pad(weight.astype(jnp.float32), ((0, EPAD - N_EDGES), (0, 0)))
    wtb = wt.astype(jnp.bfloat16)
    wtrs = [jnp.broadcast_to(wtb[h * EH:(h + 1) * EH].reshape(EH // 4, 4, 1),
                             (EH // 4, 4, 16)).reshape(EH // 4, 64)
            for h in range(2)]

    t = jnp.pad(
        jnp.concatenate([area_point[:, None], pos], axis=1).astype(jnp.float32),
        ((0, NPAD - N_NODES), (0, DT - 4)))

    convs = [
        (params['conv1'], 1, 1, 2),
        (params['conv2'], 16, 2, 2),
        (params['conv3'], 16, 2, 1),
    ]
    head = (params['W1'], params['b1'][None, :],
            params['W2'], params['b2'][None, :])

    gks = [_make_gather(0), _make_gather(1)]
    sks = [_make_scatter(0), _make_scatter(1)]
    for li, (cp, in_nf, vin, vout) in enumerate(convs):
        consts = _edge_consts(cp, in_nf, vin, vout)
        vs = []
        for h in range(2):
            gr, gc = gks[h](t, row2d, col2d)
            vs.append(_edge_tc(gr.reshape(EH // 4, 128),
                               gc.reshape(EH // 4, 128), wtrs[h], consts))
        ss = []
        for h in range(2):
            s = sks[h](vs[h].reshape(EH, DV), row2d)
            ss += [s[0], s[1]]
        t = _node_tc(t, ss, cp, in_nf, vin, vout,
                     head=head if li == 2 else None)
    return t[:N_NODES]
